# baseline (speedup 1.0000x reference)
import functools

import jax
import jax.numpy as jnp
from jax import lax
from jax.experimental import pallas as pl
from jax.experimental.pallas import tpu as pltpu
from jax.experimental.pallas import tpu_sc as plsc

_NUM_LABELS = 1000
_D = 128
_L = 16
_NC, _NS = 2, 16
_NW = _NC * _NS
_CHUNK = 128


def kernel(vertex_ids, labels, degrees, vertex_embed, label_embed, deg_W, deg_b):
    n = vertex_ids.shape[0]
    b_per_w = n // _NW
    n_chunks = b_per_w // _CHUNK
    gpc = _CHUNK // _L
    qn = _D // _L
    qh = qn // 2

    vertex_ids = vertex_ids.astype(jnp.int32)
    labels = labels.astype(jnp.int32)

    mesh = plsc.VectorSubcoreMesh(
        core_axis_name="c", subcore_axis_name="s",
        num_cores=_NC, num_subcores=_NS,
    )

    @functools.partial(
        pl.kernel,
        out_type=jax.ShapeDtypeStruct((n, _D), jnp.float32),
        mesh=mesh,
        scratch_types=[
            pltpu.VMEM((b_per_w,), jnp.int32),
            pltpu.VMEM((b_per_w,), jnp.int32),
            pltpu.VMEM((b_per_w,), jnp.float32),
            pltpu.VMEM((_D,), jnp.float32),
            pltpu.VMEM((_D,), jnp.float32),
            pltpu.VMEM((b_per_w, _D), jnp.float32),
            pltpu.SemaphoreType.DMA,
            pltpu.SemaphoreType.DMA,
        ],
    )
    def run(vid_hbm, lbl_hbm, deg_hbm, vtab_hbm, ltab_hbm, w_hbm, b_hbm,
            out_hbm, vidx, lidx, degv, wv, bv, rows, sem_in, sem_add):
        wid = lax.axis_index("s") * _NC + lax.axis_index("c")
        base = wid * b_per_w

        ins = [
            pltpu.async_copy(vid_hbm.at[pl.ds(base, b_per_w)], vidx, sem_in),
            pltpu.async_copy(lbl_hbm.at[pl.ds(base, b_per_w)], lidx, sem_in),
            pltpu.async_copy(deg_hbm.at[pl.ds(base, b_per_w)], degv, sem_in),
            pltpu.async_copy(w_hbm, wv, sem_in),
            pltpu.async_copy(b_hbm, bv, sem_in),
        ]
        for c in ins:
            c.wait()

        def fix_labels(i, _):
            lab = lidx[pl.ds(i * _L, _L)]
            lab = jnp.where((lab >= _NUM_LABELS) | (lab == -1),
                            _NUM_LABELS, lab)
            lidx[pl.ds(i * _L, _L)] = jnp.clip(lab, 0, _NUM_LABELS)
            return 0

        lax.fori_loop(0, b_per_w // _L, fix_labels, 0)

        ws = [wv[pl.ds(q * _L, _L)] for q in range(qn)]
        bs = [bv[pl.ds(q * _L, _L)] for q in range(qn)]

        def make_pass(q0):
            def half_pass(g, _):
                d16 = degv[pl.ds(g * _L, _L)]
                d16 = jnp.where(d16 * 0.0 == 0.0, d16, 1.0)
                d16 = jnp.maximum(d16, 1.0)
                for k in range(_L):
                    d = jnp.full((_L,), d16[k], jnp.float32)
                    i = g * _L + k
                    for q in range(q0, q0 + qh):
                        rows[i, pl.ds(q * _L, _L)] = d * ws[q] + bs[q]
                return 0
            return half_pass

        pass_lo = make_pass(0)
        pass_hi = make_pass(qh)

        def chunk_body(j, _):
            lax.fori_loop(j * gpc, (j + 1) * gpc, pass_lo, 0)
            lax.fori_loop(j * gpc, (j + 1) * gpc, pass_hi, 0)
            dst = rows.at[pl.ds(j * _CHUNK, _CHUNK)]
            vsl = vidx.at[pl.ds(j * _CHUNK, _CHUNK)]
            lsl = lidx.at[pl.ds(j * _CHUNK, _CHUNK)]
            pltpu.async_copy(vtab_hbm.at[vsl], dst, sem_add, add=True)
            pltpu.async_copy(ltab_hbm.at[lsl], dst, sem_add, add=True)
            return 0

        lax.fori_loop(0, n_chunks, chunk_body, 0)

        def drain_body(j, _):
            dst = rows.at[pl.ds(j * _CHUNK, _CHUNK)]
            vsl = vidx.at[pl.ds(j * _CHUNK, _CHUNK)]
            lsl = lidx.at[pl.ds(j * _CHUNK, _CHUNK)]
            pltpu.make_async_copy(vtab_hbm.at[vsl], dst, sem_add).wait()
            pltpu.make_async_copy(ltab_hbm.at[lsl], dst, sem_add).wait()
            return 0

        lax.fori_loop(0, n_chunks, drain_body, 0)

        pltpu.sync_copy(rows, out_hbm.at[pl.ds(base, b_per_w)])

    return run(vertex_ids, labels, degrees, vertex_embed, label_embed,
               deg_W, deg_b)
